# Initial kernel scaffold; baseline (speedup 1.0000x reference)
import jax
import jax.numpy as jnp
from jax.experimental import pallas as pl


def kernel(node_features, edge_indices, edge_features, Wn, bn, We, be, Wt, bt, Wg0, bg0, Wg1, bg1, Wg2, bg2, ln_s0, ln_b0, ln_s1, ln_b1, ln_s2, ln_b2):
    raise NotImplementedError("write your pallas kernel here")


# speedup vs baseline: 4.3730x; 4.3730x over previous
import functools

import jax
import jax.numpy as jnp
from jax import lax
from jax.experimental import pallas as pl
from jax.experimental.pallas import tpu as pltpu
from jax.experimental.pallas import tpu_sc as plsc

N = 10000
E = 320000
D_NODE = 128
D_EDGE = 16
H = 128
EPS = 1e-6

NC = 2
NS = 16
NW = NC * NS
EPT = E // NW
C = 80
NCHUNK = EPT // C
NPAD = 10240
NPT = NPAD // NS
ZR = 128

ROW_BLK = 400


def _tc_init(interpret=False):
    def body(x_ref, wn_ref, bn_ref, wtf_ref, h_ref, t2_ref):
        h = jnp.dot(x_ref[...], wn_ref[...], preferred_element_type=jnp.float32, precision=lax.Precision.HIGHEST)
        h = h + bn_ref[...][None, :]
        h_ref[...] = h
        wt1 = wtf_ref[0:H]
        wt2 = wtf_ref[H:2 * H]
        ts = jnp.dot(h, wt1, preferred_element_type=jnp.float32, precision=lax.Precision.HIGHEST)
        td = jnp.dot(h, wt2, preferred_element_type=jnp.float32, precision=lax.Precision.HIGHEST)
        t2_ref[...] = jnp.stack([ts, td], axis=1)

    grid = (N // ROW_BLK,)
    return pl.pallas_call(
        body,
        grid=grid,
        in_specs=[
            pl.BlockSpec((ROW_BLK, D_NODE), lambda i: (i, 0)),
            pl.BlockSpec((D_NODE, H), lambda i: (0, 0)),
            pl.BlockSpec((H,), lambda i: (0,)),
            pl.BlockSpec((3 * H,), lambda i: (0,)),
        ],
        out_specs=[
            pl.BlockSpec((ROW_BLK, H), lambda i: (i, 0)),
            pl.BlockSpec((ROW_BLK, 2), lambda i: (i, 0)),
        ],
        out_shape=[
            jax.ShapeDtypeStruct((N, H), jnp.float32),
            jax.ShapeDtypeStruct((N, 2), jnp.float32),
        ],
        interpret=interpret,
    )


def _tc_update(interpret=False):
    def body(h_ref, aw_ref, ea_ref, we_ref, be_ref, wg_ref, bg_ref,
             ls_ref, lb_ref, wtf_ref, o_ref, t2_ref):
        h = h_ref[...]
        wg = wg_ref[...]
        wga = wg[0:H, :]
        wgb = wg[H:2 * H, :]
        aw = aw_ref[0] + aw_ref[1]
        a16 = ea_ref[0, :, 0:D_EDGE] + ea_ref[1, :, 0:D_EDGE]
        deg = ea_ref[0, :, D_EDGE:D_EDGE + 1] + ea_ref[1, :, D_EDGE:D_EDGE + 1]
        wewgb = jnp.dot(we_ref[...], wgb, preferred_element_type=jnp.float32, precision=lax.Precision.HIGHEST)
        bevec = jnp.dot(be_ref[...], wgb, preferred_element_type=jnp.float32, precision=lax.Precision.HIGHEST)
        u = jnp.dot(h, wga, preferred_element_type=jnp.float32, precision=lax.Precision.HIGHEST)
        u = u + jnp.dot(aw, wgb, preferred_element_type=jnp.float32, precision=lax.Precision.HIGHEST)
        u = u + jnp.dot(a16, wewgb, preferred_element_type=jnp.float32, precision=lax.Precision.HIGHEST)
        u = u + deg * bevec[None, :]
        u = u + bg_ref[...][None, :]
        r = h + u
        mean = jnp.mean(r, axis=-1, keepdims=True)
        var = jnp.mean(jnp.square(r - mean), axis=-1, keepdims=True)
        y = (r - mean) * lax.rsqrt(var + EPS) * ls_ref[...][None, :] + lb_ref[...][None, :]
        hn = jnp.maximum(y, 0.0)
        o_ref[...] = hn
        wt1 = wtf_ref[0:H]
        wt2 = wtf_ref[H:2 * H]
        ts = jnp.dot(hn, wt1, preferred_element_type=jnp.float32, precision=lax.Precision.HIGHEST)
        td = jnp.dot(hn, wt2, preferred_element_type=jnp.float32, precision=lax.Precision.HIGHEST)
        t2_ref[...] = jnp.stack([ts, td], axis=1)

    grid = (N // ROW_BLK,)
    return pl.pallas_call(
        body,
        grid=grid,
        in_specs=[
            pl.BlockSpec((ROW_BLK, H), lambda i: (i, 0)),
            pl.BlockSpec((2, ROW_BLK, H), lambda i: (0, i, 0)),
            pl.BlockSpec((2, ROW_BLK, 2 * D_EDGE), lambda i: (0, i, 0)),
            pl.BlockSpec((D_EDGE, H), lambda i: (0, 0)),
            pl.BlockSpec((H,), lambda i: (0,)),
            pl.BlockSpec((2 * H, H), lambda i: (0, 0)),
            pl.BlockSpec((H,), lambda i: (0,)),
            pl.BlockSpec((H,), lambda i: (0,)),
            pl.BlockSpec((H,), lambda i: (0,)),
            pl.BlockSpec((3 * H,), lambda i: (0,)),
        ],
        out_specs=[
            pl.BlockSpec((ROW_BLK, H), lambda i: (i, 0)),
            pl.BlockSpec((ROW_BLK, 2), lambda i: (i, 0)),
        ],
        out_shape=[
            jax.ShapeDtypeStruct((N, H), jnp.float32),
            jax.ShapeDtypeStruct((N, 2), jnp.float32),
        ],
        interpret=interpret,
    )


def _sc_edge_pre(interpret=False):
    mesh = plsc.VectorSubcoreMesh(core_axis_name="c", subcore_axis_name="s", num_cores=NC, num_subcores=NS)

    @functools.partial(
        pl.kernel,
        mesh=mesh,
        compiler_params=pltpu.CompilerParams(use_tc_tiling_on_sc=False, needs_layout_passes=False),
        out_type=[
            jax.ShapeDtypeStruct((E,), jnp.float32),
            jax.ShapeDtypeStruct((NC, NPAD, 2 * D_EDGE), jnp.float32),
        ],
        scratch_types=[
            pltpu.VMEM((C, D_EDGE), jnp.float32),
            pltpu.VMEM((C, 2 * D_EDGE), jnp.float32),
            pltpu.VMEM((EPT,), jnp.int32),
            pltpu.VMEM((C,), jnp.int32),
            pltpu.VMEM((EPT,), jnp.float32),
            pltpu.VMEM((D_EDGE, H), jnp.float32),
            pltpu.VMEM((3 * H,), jnp.float32),
            pltpu.VMEM((16,), jnp.float32),
            pltpu.VMEM((H,), jnp.float32),
            pltpu.VMEM((ZR, 2 * D_EDGE), jnp.float32),
            pltpu.VMEM_SHARED((NPAD, 2 * D_EDGE), jnp.float32),
            pltpu.SemaphoreType.DMA,
        ],
        interpret=interpret,
    )
    def k(ef_hbm, dst_hbm, we_hbm, wtf_hbm, bt_hbm, be_hbm, z_hbm,
          te_hbm, eagg_hbm,
          ef_c, msg, dstf, dst80, tef, we_v, wt_v, bt_v, be_v, zbuf,
          eshared, sem):
        cid = lax.axis_index("c")
        sid = lax.axis_index("s")
        wid = sid * NC + cid
        base = pl.multiple_of(wid * EPT, 8)

        pltpu.sync_copy(z_hbm, zbuf)
        for r in range(NPT // ZR):
            pltpu.sync_copy(zbuf, eshared.at[pl.ds(sid * NPT + r * ZR, ZR)])
        plsc.subcore_barrier()

        pltpu.sync_copy(dst_hbm.at[pl.ds(base, EPT)], dstf)
        pltpu.sync_copy(we_hbm, we_v)
        pltpu.sync_copy(wtf_hbm, wt_v)
        pltpu.sync_copy(bt_hbm, bt_v)
        pltpu.sync_copy(be_hbm, be_v)

        lanes = lax.iota(jnp.int32, 16)
        q16 = jnp.zeros((16,), jnp.float32)
        for kk in range(D_EDGE):
            acc = jnp.zeros((16,), jnp.float32)
            for j in range(H // 16):
                acc = acc + we_v[kk, pl.ds(j * 16, 16)] * wt_v[pl.ds(2 * H + j * 16, 16)]
            s = jnp.sum(acc)
            q16 = jnp.where(lanes == kk, s, q16)
        acc = jnp.zeros((16,), jnp.float32)
        for j in range(H // 16):
            acc = acc + be_v[pl.ds(j * 16, 16)] * wt_v[pl.ds(2 * H + j * 16, 16)]
        c_sc = jnp.sum(acc) + jnp.max(bt_v[...])

        onehot = jnp.where(lanes == 0, 1.0, 0.0).astype(jnp.float32)
        for e in range(C):
            msg[e, pl.ds(D_EDGE, 16)] = onehot

        def chunk(kc, _):
            cb = kc * C
            pltpu.sync_copy(ef_hbm.at[pl.ds(base + cb, C), :], ef_c)
            for g in range(C // 16):
                dst80[pl.ds(g * 16, 16)] = dstf[pl.ds(cb + g * 16, 16)]
            for g in range(C // 16):
                te16 = jnp.zeros((16,), jnp.float32)
                for t in range(16):
                    e = g * 16 + t
                    v = ef_c[e, :]
                    msg[e, pl.ds(0, 16)] = v
                    s = jnp.sum(v * q16) + c_sc
                    te16 = jnp.where(lanes == t, s, te16)
                tef[pl.ds(cb + g * 16, 16)] = te16
            pltpu.sync_copy(msg, eshared.at[dst80], add=True)
            return 0

        lax.fori_loop(0, NCHUNK, chunk, 0)
        pltpu.sync_copy(tef, te_hbm.at[pl.ds(base, EPT)])

        plsc.subcore_barrier()
        for r in range(NPT // ZR):
            rb = sid * NPT + r * ZR
            pltpu.sync_copy(eshared.at[pl.ds(rb, ZR)],
                            eagg_hbm.at[cid, pl.ds(rb, ZR)])

    return k


def _sc_layer(interpret=False):
    mesh = plsc.VectorSubcoreMesh(core_axis_name="c", subcore_axis_name="s", num_cores=NC, num_subcores=NS)

    @functools.partial(
        pl.kernel,
        mesh=mesh,
        compiler_params=pltpu.CompilerParams(use_tc_tiling_on_sc=False, needs_layout_passes=False),
        out_type=[
            jax.ShapeDtypeStruct((E,), jnp.float32),
            jax.ShapeDtypeStruct((NC, NPAD, H), jnp.float32),
        ],
        scratch_types=[
            pltpu.VMEM((2 * N,), jnp.float32),
            pltpu.VMEM((C,), jnp.float32),
            pltpu.VMEM((C,), jnp.int32),
            pltpu.VMEM((C,), jnp.int32),
            pltpu.VMEM((C,), jnp.float32),
            pltpu.VMEM((C, H), jnp.float32),
            pltpu.VMEM_SHARED((NPAD, H), jnp.float32),
            pltpu.SemaphoreType.DMA,
        ],
        interpret=interpret,
    )
    def k(nh_hbm, t2_hbm, te_hbm, src_hbm, dst_hbm, z_hbm,
          w_hbm, aggw_hbm,
          t2v, te80, src80, dst80, w80, rows,
          ashared, sem):
        cid = lax.axis_index("c")
        sid = lax.axis_index("s")
        wid = sid * NC + cid
        base = pl.multiple_of(wid * EPT, 8)

        pltpu.sync_copy(z_hbm, rows)
        for r in range(NPT // C):
            pltpu.sync_copy(rows, ashared.at[pl.ds(sid * NPT + r * C, C)])
        pltpu.sync_copy(t2_hbm, t2v)
        plsc.subcore_barrier()

        def chunk(kc, _):
            cb = pl.multiple_of(base + kc * C, 8)
            pltpu.sync_copy(src_hbm.at[pl.ds(cb, C)], src80)
            pltpu.sync_copy(dst_hbm.at[pl.ds(cb, C)], dst80)
            pltpu.sync_copy(te_hbm.at[pl.ds(cb, C)], te80)
            pltpu.async_copy(nh_hbm.at[src80], rows, sem).wait()
            for g in range(C // 16):
                si = src80[pl.ds(g * 16, 16)]
                di = dst80[pl.ds(g * 16, 16)]
                ts = plsc.load_gather(t2v, [si * 2])
                td = plsc.load_gather(t2v, [di * 2 + 1])
                x = ts + td + te80[pl.ds(g * 16, 16)]
                w16 = 1.0 / (1.0 + jnp.exp(-x))
                w80[pl.ds(g * 16, 16)] = w16
                for t in range(16):
                    e = g * 16 + t
                    wv = w16[t]
                    for j in range(H // 16):
                        rows[e, pl.ds(j * 16, 16)] = rows[e, pl.ds(j * 16, 16)] * wv
            pltpu.sync_copy(w80, w_hbm.at[pl.ds(cb, C)])
            pltpu.sync_copy(rows, ashared.at[dst80], add=True)
            return 0

        lax.fori_loop(0, NCHUNK, chunk, 0)

        plsc.subcore_barrier()
        for r in range(NPT // C):
            rb = sid * NPT + r * C
            pltpu.sync_copy(ashared.at[pl.ds(rb, C)],
                            aggw_hbm.at[cid, pl.ds(rb, C)])

    return k


def _make_kernel(interpret=False):
    tc_init = _tc_init(interpret)
    tc_update = _tc_update(interpret)
    sc_pre = _sc_edge_pre(interpret)
    sc_layer = _sc_layer(interpret)

    def run(node_features, edge_indices, edge_features, Wn, bn, We, be, Wt, bt,
            Wg0, bg0, Wg1, bg1, Wg2, bg2,
            ln_s0, ln_b0, ln_s1, ln_b1, ln_s2, ln_b2):
        src = edge_indices[0]
        dst = edge_indices[1]
        wtf = Wt.reshape(3 * H)
        bt16 = jnp.broadcast_to(bt, (16,))
        z32 = jnp.zeros((ZR, 2 * D_EDGE), jnp.float32)
        z128 = jnp.zeros((C, H), jnp.float32)

        h, t2 = tc_init(node_features, Wn, bn, wtf)
        t_edge, eagg = sc_pre(edge_features, dst, We, wtf, bt16, be, z32)

        Wgs = [Wg0, Wg1, Wg2]
        bgs = [bg0, bg1, bg2]
        lss = [ln_s0, ln_s1, ln_s2]
        lbs = [ln_b0, ln_b1, ln_b2]
        w = None
        for i in range(3):
            w, aggw = sc_layer(h, t2.reshape(2 * N), t_edge, src, dst, z128)
            h, t2 = tc_update(h, aggw, eagg, We, be, Wgs[i], bgs[i],
                              lss[i], lbs[i], wtf)
        return h, w

    return run


kernel = _make_kernel()
